# baseline (speedup 1.0000x reference)
import functools

import jax
import jax.numpy as jnp
from jax import lax
from jax.experimental import pallas as pl
from jax.experimental.pallas import tpu as pltpu
from jax.experimental.pallas import tpu_sc as plsc

N_ATOM = 10000
N_BOND = 160000
E_BB = 640000
D_ATOM = 128
D_BOND = 16
H = 128
N_MOL = 256

NC = 2
NS = 16
NW = NC * NS

NB_PAD = 163840

@functools.cache
def _sc_mesh():
    return plsc.VectorSubcoreMesh(
        core_axis_name="c", subcore_axis_name="s", num_cores=NC, num_subcores=NS
    )


def _tc_matmul_body(x_ref, w_ref, o_ref):
    o_ref[...] = jnp.dot(x_ref[...], w_ref[...], preferred_element_type=jnp.float32)


def _tc_matmul(x, w, block_rows):
    m, k = x.shape
    n = w.shape[1]
    grid = m // block_rows
    return pl.pallas_call(
        _tc_matmul_body,
        grid=(grid,),
        in_specs=[
            pl.BlockSpec((block_rows, k), lambda i: (i, 0)),
            pl.BlockSpec((k, n), lambda i: (0, 0)),
        ],
        out_specs=pl.BlockSpec((block_rows, n), lambda i: (i, 0)),
        out_shape=jax.ShapeDtypeStruct((m, n), jnp.float32),
        interpret=False,
    )(x, w)


def _tc_bond_proj(e2, w_blk):
    return pl.pallas_call(
        _tc_matmul_body,
        grid=(8,),
        in_specs=[
            pl.BlockSpec((2560, 128), lambda i: (i, 0)),
            pl.BlockSpec((128, 1024), lambda i: (0, 0)),
        ],
        out_specs=pl.BlockSpec((2560, 1024), lambda i: (i, 0)),
        out_shape=jax.ShapeDtypeStruct((20480, 1024), jnp.float32),
        interpret=False,
    )(e2, w_blk)


def _tc_relu_body(x_ref, o_ref):
    o_ref[...] = jnp.maximum(x_ref[...], 0.0)


def _tc_relu(x):
    m, n = x.shape
    return pl.pallas_call(
        _tc_relu_body,
        grid=(16,),
        in_specs=[pl.BlockSpec((m // 16, n), lambda i: (i, 0))],
        out_specs=pl.BlockSpec((m // 16, n), lambda i: (i, 0)),
        out_shape=jax.ShapeDtypeStruct((m, n), jnp.float32),
        interpret=False,
    )(x)


def _tc_h1_body(h0_ref, m_ref, w_ref, o_ref):
    mm = jnp.dot(m_ref[...], w_ref[...], preferred_element_type=jnp.float32)
    o_ref[...] = jnp.maximum(h0_ref[...] + mm, 0.0)


def _tc_h1(h0, m, w_h):
    return pl.pallas_call(
        _tc_h1_body,
        grid=(32,),
        in_specs=[
            pl.BlockSpec((5120, H), lambda i: (i, 0)),
            pl.BlockSpec((5120, H), lambda i: (i, 0)),
            pl.BlockSpec((H, H), lambda i: (0, 0)),
        ],
        out_specs=pl.BlockSpec((5120, H), lambda i: (i, 0)),
        out_shape=jax.ShapeDtypeStruct((NB_PAD, H), jnp.float32),
        interpret=False,
    )(h0, m, w_h)


def _tc_final_body(ax_ref, mv_ref, wo1_ref, wo2_ref, b_ref, mol_ref, o_ref, acc_ref):
    i = pl.program_id(0)
    hv = jnp.dot(ax_ref[...], wo1_ref[...], preferred_element_type=jnp.float32)
    hv += jnp.dot(mv_ref[...], wo2_ref[...], preferred_element_type=jnp.float32)
    hv = jnp.maximum(hv + b_ref[...], 0.0)
    blk = hv.shape[0]
    col = lax.broadcasted_iota(jnp.int32, (blk, H), 1)
    ones_col = jnp.where(col == 0, 1.0, 0.0)
    hv_ext = jnp.concatenate([hv, ones_col], axis=1)
    mol = mol_ref[0, 0, :]
    seg = lax.broadcasted_iota(jnp.int32, (N_MOL, blk), 0)
    onehot = jnp.where(seg == mol[None, :], 1.0, 0.0)
    part = lax.dot_general(
        onehot, hv_ext, (((1,), (0,)), ((), ())), preferred_element_type=jnp.float32
    )

    @pl.when(i == 0)
    def _init():
        acc_ref[...] = jnp.zeros_like(acc_ref)

    acc_ref[...] += part

    @pl.when(i == pl.num_programs(0) - 1)
    def _fin():
        acc = acc_ref[...]
        sums = acc[:, :H]
        counts = acc[:, H : H + 1]
        o_ref[...] = sums / jnp.maximum(counts, 1.0)


def _tc_final(atom_x, mv, wo1, wo2, b2, mol3d):
    blk = 2000
    return pl.pallas_call(
        _tc_final_body,
        grid=(N_ATOM // blk,),
        in_specs=[
            pl.BlockSpec((blk, D_ATOM), lambda i: (i, 0)),
            pl.BlockSpec((blk, H), lambda i: (i, 0)),
            pl.BlockSpec((D_ATOM, H), lambda i: (0, 0)),
            pl.BlockSpec((H, H), lambda i: (0, 0)),
            pl.BlockSpec((1, H), lambda i: (0, 0)),
            pl.BlockSpec((1, 1, blk), lambda i: (i, 0, 0)),
        ],
        out_specs=pl.BlockSpec((N_MOL, H), lambda i: (0, 0)),
        out_shape=jax.ShapeDtypeStruct((N_MOL, H), jnp.float32),
        scratch_shapes=[pltpu.VMEM((N_MOL, 2 * H), jnp.float32)],
        interpret=False,
    )(atom_x, mv, wo1, wo2, b2, mol3d)


_ROWS_PER_W = NB_PAD // NW
_IDXROWS_PER_W = _ROWS_PER_W // 128
_EB = 512
_NJ = _EB // 128


def _sc_gather_add_body(p_hbm, bm_hbm, src_hbm, z_hbm, rowbuf, idxv, sem):
    wid = lax.axis_index("s") * NC + lax.axis_index("c")
    row0 = wid * _ROWS_PER_W
    irow0 = wid * _IDXROWS_PER_W

    def body(b, carry):
        pltpu.sync_copy(src_hbm.at[pl.ds(irow0 + b * 8, 8)], idxv)
        for jj in range(2):
            base = row0 + (b * 2 + jj) * _EB
            pltpu.sync_copy(bm_hbm.at[pl.ds(base, _EB)], rowbuf)
            descs = []
            for j in range(_NJ):
                descs.append(
                    pltpu.async_copy(
                        p_hbm.at[idxv.at[jj * _NJ + j]],
                        rowbuf.at[pl.ds(j * 128, 128)],
                        sem,
                        add=True,
                    )
                )
            for d in descs:
                d.wait()
            pltpu.sync_copy(rowbuf, z_hbm.at[pl.ds(base, _EB)])
        return carry

    lax.fori_loop(0, _IDXROWS_PER_W // 8, body, 0)


@functools.cache
def _make_sc_gather_add():
    return pl.kernel(
        _sc_gather_add_body,
        out_type=jax.ShapeDtypeStruct((NB_PAD, H), jnp.float32),
        mesh=_sc_mesh(),
        scratch_types=[
            pltpu.VMEM((_EB, H), jnp.float32),
            pltpu.VMEM((8, 128), jnp.int32),
            pltpu.SemaphoreType.DMA,
        ],
        interpret=False,
    )


_CH_SC = 4096
_CH = _CH_SC * NC
_NCHUNK = NB_PAD // _CH
_GARB = _CH_SC
_EROWS_PAD = 5120
_EROWS_PER_T = _EROWS_PAD // NS
_DRAIN = _CH_SC // NS


def _zero_zbuf(zbuf, nrows):
    def zrow(r, carry):
        for cc in range(8):
            zbuf[r, pl.ds(cc * 16, 16)] = jnp.zeros((16,), jnp.float32)
        return carry

    lax.fori_loop(0, nrows, zrow, 0)


def _sc_segsum_bb_body(
    h_hbm, bbs_hbm, bbd_hbm, m_hbm, rowbuf, srcv, lidx, zbuf, acc, sem
):
    k = lax.axis_index("c")
    s = lax.axis_index("s")
    erow0 = s * _EROWS_PER_T

    _zero_zbuf(zbuf, 128)
    for c in range(_NCHUNK):
        lo = c * _CH + k * _CH_SC
        for zz in range(_DRAIN // 128):
            pltpu.sync_copy(zbuf, acc.at[pl.ds(s * _DRAIN + zz * 128, 128)])
        plsc.subcore_barrier()

        def body(b, carry):
            er = erow0 + b * 8
            pltpu.sync_copy(bbd_hbm.at[pl.ds(er, 8)], lidx)
            pltpu.sync_copy(bbs_hbm.at[pl.ds(er, 8)], srcv)
            for j in range(8):
                for cc in range(8):
                    dd = lidx[j, pl.ds(cc * 16, 16)]
                    keep = (dd >= lo) & (dd < lo + _CH_SC)
                    lidx[j, pl.ds(cc * 16, 16)] = jnp.where(keep, dd - lo, _GARB)
            for jj in range(2):
                descs = []
                for j in range(_NJ):
                    descs.append(
                        pltpu.async_copy(
                            h_hbm.at[srcv.at[jj * _NJ + j]],
                            rowbuf.at[pl.ds(j * 128, 128)],
                            sem,
                        )
                    )
                for d in descs:
                    d.wait()
                for j in range(_NJ):
                    pltpu.sync_copy(
                        rowbuf.at[pl.ds(j * 128, 128)],
                        acc.at[lidx.at[jj * _NJ + j]],
                        add=True,
                    )
            return carry

        lax.fori_loop(0, _EROWS_PER_T // 8, body, 0)
        plsc.subcore_barrier()
        drain0 = c * _CH + k * _CH_SC + s * _DRAIN
        pltpu.sync_copy(
            acc.at[pl.ds(s * _DRAIN, _DRAIN)], m_hbm.at[pl.ds(drain0, _DRAIN)]
        )
        plsc.subcore_barrier()


@functools.cache
def _make_sc_segsum_bb():
    return pl.kernel(
        _sc_segsum_bb_body,
        out_type=jax.ShapeDtypeStruct((NB_PAD, H), jnp.float32),
        mesh=_sc_mesh(),
        scratch_types=[
            pltpu.VMEM((_EB, H), jnp.float32),
            pltpu.VMEM((8, 128), jnp.int32),
            pltpu.VMEM((8, 128), jnp.int32),
            pltpu.VMEM((128, H), jnp.float32),
            pltpu.VMEM_SHARED((_CH_SC + 8, H), jnp.float32),
            pltpu.SemaphoreType.DMA,
        ],
        interpret=False,
    )


_AT_SC = 5120
_AT_PAD = _AT_SC * NC
_AT_DRAIN = _AT_SC // NS
_AT_ROWS_PER_T = NB_PAD // NS
_AT_IDXROWS_PER_T = _AT_ROWS_PER_T // 128


def _sc_segsum_atom_body(h_hbm, ba_hbm, mv_hbm, rowbuf, idxv, zbuf, acc, sem):
    k = lax.axis_index("c")
    s = lax.axis_index("s")
    row0 = s * _AT_ROWS_PER_T
    irow0 = s * _AT_IDXROWS_PER_T
    lo = k * _AT_SC

    _zero_zbuf(zbuf, 80)
    for zz in range(_AT_DRAIN // 80):
        pltpu.sync_copy(zbuf, acc.at[pl.ds(s * _AT_DRAIN + zz * 80, 80)])
    plsc.subcore_barrier()

    def body(b, carry):
        pltpu.sync_copy(ba_hbm.at[pl.ds(irow0 + b * 8, 8)], idxv)
        for j in range(8):
            for cc in range(8):
                a = idxv[j, pl.ds(cc * 16, 16)]
                keep = (a >= lo) & (a < lo + _AT_SC)
                idxv[j, pl.ds(cc * 16, 16)] = jnp.where(keep, a - lo, _AT_SC)
        for jj in range(2):
            base = row0 + (b * 2 + jj) * _EB
            pltpu.sync_copy(h_hbm.at[pl.ds(base, _EB)], rowbuf)
            for j in range(_NJ):
                pltpu.sync_copy(
                    rowbuf.at[pl.ds(j * 128, 128)],
                    acc.at[idxv.at[jj * _NJ + j]],
                    add=True,
                )
        return carry

    lax.fori_loop(0, _AT_IDXROWS_PER_T // 8, body, 0)
    plsc.subcore_barrier()
    pltpu.sync_copy(
        acc.at[pl.ds(s * _AT_DRAIN, _AT_DRAIN)],
        mv_hbm.at[pl.ds(lo + s * _AT_DRAIN, _AT_DRAIN)],
    )


@functools.cache
def _make_sc_segsum_atom():
    return pl.kernel(
        _sc_segsum_atom_body,
        out_type=jax.ShapeDtypeStruct((_AT_PAD, H), jnp.float32),
        mesh=_sc_mesh(),
        scratch_types=[
            pltpu.VMEM((_EB, H), jnp.float32),
            pltpu.VMEM((8, 128), jnp.int32),
            pltpu.VMEM((80, H), jnp.float32),
            pltpu.VMEM_SHARED((_AT_SC + 8, H), jnp.float32),
            pltpu.SemaphoreType.DMA,
        ],
        interpret=False,
    )


def kernel(atom_x, bond_e, W_i, W_h, W_o_w, W_o_b, src_atom, bb_src, bb_dst, bond_atom, mol_id):
    wa = W_i[:D_ATOM]
    wb = W_i[D_ATOM:]
    w_blk = jnp.zeros((128, 1024), jnp.float32)
    for j in range(8):
        w_blk = lax.dynamic_update_slice(w_blk, wb, (16 * j, 128 * j))

    src2d = jnp.pad(src_atom, (0, NB_PAD - N_BOND)).reshape(1280, 128)
    ba2d = jnp.pad(bond_atom, (0, NB_PAD - N_BOND), constant_values=N_ATOM).reshape(
        1280, 128
    )
    bbs2d = jnp.pad(bb_src, (0, _EROWS_PAD * 128 - E_BB)).reshape(_EROWS_PAD, 128)
    bbd2d = jnp.pad(
        bb_dst, (0, _EROWS_PAD * 128 - E_BB), constant_values=N_BOND
    ).reshape(_EROWS_PAD, 128)

    p = _tc_matmul(atom_x, wa, 2000)
    e2 = bond_e.reshape(20000, 128)
    bm = _tc_bond_proj(e2, w_blk).reshape(NB_PAD, H)

    z0 = _make_sc_gather_add()(p, bm, src2d)
    h0 = _tc_relu(z0)
    m = _make_sc_segsum_bb()(h0, bbs2d, bbd2d)
    h1 = _tc_h1(h0, m, W_h)
    mv = _make_sc_segsum_atom()(h1, ba2d)
    mol3d = mol_id.reshape(5, 1, 2000)
    out = _tc_final(
        atom_x, mv, W_o_w[:D_ATOM], W_o_w[D_ATOM:], W_o_b.reshape(1, H), mol3d
    )
    return out


# speedup vs baseline: 3.9260x; 3.9260x over previous
import functools

import jax
import jax.numpy as jnp
from jax import lax
from jax.experimental import pallas as pl
from jax.experimental.pallas import tpu as pltpu
from jax.experimental.pallas import tpu_sc as plsc

N_ATOM = 10000
N_BOND = 160000
E_BB = 640000
D_ATOM = 128
D_BOND = 16
H = 128
N_MOL = 256

NC = 2
NS = 16
NW = NC * NS

NB_PAD = 163840

@functools.cache
def _sc_mesh():
    return plsc.VectorSubcoreMesh(
        core_axis_name="c", subcore_axis_name="s", num_cores=NC, num_subcores=NS
    )


def _tc_matmul_body(x_ref, w_ref, o_ref):
    o_ref[...] = jnp.dot(x_ref[...], w_ref[...], preferred_element_type=jnp.float32)


def _tc_matmul(x, w, block_rows):
    m, k = x.shape
    n = w.shape[1]
    grid = m // block_rows
    return pl.pallas_call(
        _tc_matmul_body,
        grid=(grid,),
        in_specs=[
            pl.BlockSpec((block_rows, k), lambda i: (i, 0)),
            pl.BlockSpec((k, n), lambda i: (0, 0)),
        ],
        out_specs=pl.BlockSpec((block_rows, n), lambda i: (i, 0)),
        out_shape=jax.ShapeDtypeStruct((m, n), jnp.float32),
        interpret=False,
    )(x, w)


def _tc_bond_proj(e2, w_blk):
    return pl.pallas_call(
        _tc_matmul_body,
        grid=(8,),
        in_specs=[
            pl.BlockSpec((2560, 128), lambda i: (i, 0)),
            pl.BlockSpec((128, 1024), lambda i: (0, 0)),
        ],
        out_specs=pl.BlockSpec((2560, 1024), lambda i: (i, 0)),
        out_shape=jax.ShapeDtypeStruct((20480, 1024), jnp.float32),
        interpret=False,
    )(e2, w_blk)


def _tc_relu_body(x_ref, o_ref):
    o_ref[...] = jnp.maximum(x_ref[...], 0.0)


def _tc_relu(x):
    m, n = x.shape
    return pl.pallas_call(
        _tc_relu_body,
        grid=(16,),
        in_specs=[pl.BlockSpec((m // 16, n), lambda i: (i, 0))],
        out_specs=pl.BlockSpec((m // 16, n), lambda i: (i, 0)),
        out_shape=jax.ShapeDtypeStruct((m, n), jnp.float32),
        interpret=False,
    )(x)


def _tc_h1_body(h0_ref, m_ref, w_ref, o_ref):
    mm = jnp.dot(m_ref[...], w_ref[...], preferred_element_type=jnp.float32)
    o_ref[...] = jnp.maximum(h0_ref[...] + mm, 0.0)


def _tc_h1(h0, m, w_h):
    return pl.pallas_call(
        _tc_h1_body,
        grid=(32,),
        in_specs=[
            pl.BlockSpec((5120, H), lambda i: (i, 0)),
            pl.BlockSpec((5120, H), lambda i: (i, 0)),
            pl.BlockSpec((H, H), lambda i: (0, 0)),
        ],
        out_specs=pl.BlockSpec((5120, H), lambda i: (i, 0)),
        out_shape=jax.ShapeDtypeStruct((NB_PAD, H), jnp.float32),
        interpret=False,
    )(h0, m, w_h)


def _tc_final_body(ax_ref, mv_ref, wo1_ref, wo2_ref, b_ref, mol_ref, o_ref, acc_ref):
    i = pl.program_id(0)
    hv = jnp.dot(ax_ref[...], wo1_ref[...], preferred_element_type=jnp.float32)
    hv += jnp.dot(mv_ref[...], wo2_ref[...], preferred_element_type=jnp.float32)
    hv = jnp.maximum(hv + b_ref[...], 0.0)
    blk = hv.shape[0]
    col = lax.broadcasted_iota(jnp.int32, (blk, H), 1)
    ones_col = jnp.where(col == 0, 1.0, 0.0)
    hv_ext = jnp.concatenate([hv, ones_col], axis=1)
    mol = mol_ref[0, 0, :]
    seg = lax.broadcasted_iota(jnp.int32, (N_MOL, blk), 0)
    onehot = jnp.where(seg == mol[None, :], 1.0, 0.0)
    part = lax.dot_general(
        onehot, hv_ext, (((1,), (0,)), ((), ())), preferred_element_type=jnp.float32
    )

    @pl.when(i == 0)
    def _init():
        acc_ref[...] = jnp.zeros_like(acc_ref)

    acc_ref[...] += part

    @pl.when(i == pl.num_programs(0) - 1)
    def _fin():
        acc = acc_ref[...]
        sums = acc[:, :H]
        counts = acc[:, H : H + 1]
        o_ref[...] = sums / jnp.maximum(counts, 1.0)


def _tc_final(atom_x, mv, wo1, wo2, b2, mol3d):
    blk = 2000
    return pl.pallas_call(
        _tc_final_body,
        grid=(N_ATOM // blk,),
        in_specs=[
            pl.BlockSpec((blk, D_ATOM), lambda i: (i, 0)),
            pl.BlockSpec((blk, H), lambda i: (i, 0)),
            pl.BlockSpec((D_ATOM, H), lambda i: (0, 0)),
            pl.BlockSpec((H, H), lambda i: (0, 0)),
            pl.BlockSpec((1, H), lambda i: (0, 0)),
            pl.BlockSpec((1, 1, blk), lambda i: (i, 0, 0)),
        ],
        out_specs=pl.BlockSpec((N_MOL, H), lambda i: (0, 0)),
        out_shape=jax.ShapeDtypeStruct((N_MOL, H), jnp.float32),
        scratch_shapes=[pltpu.VMEM((N_MOL, 2 * H), jnp.float32)],
        interpret=False,
    )(atom_x, mv, wo1, wo2, b2, mol3d)


_ROWS_PER_W = NB_PAD // NW
_IDXROWS_PER_W = _ROWS_PER_W // 128
_EB = 512
_NJ = _EB // 128


def _sc_gather_add_body(p_hbm, bm_hbm, src_hbm, z_hbm, rowbuf, idxv, sem):
    wid = lax.axis_index("s") * NC + lax.axis_index("c")
    row0 = wid * _ROWS_PER_W
    irow0 = wid * _IDXROWS_PER_W

    def body(b, carry):
        pltpu.sync_copy(src_hbm.at[pl.ds(irow0 + b * 8, 8)], idxv)
        for jj in range(2):
            base = row0 + (b * 2 + jj) * _EB
            pltpu.sync_copy(bm_hbm.at[pl.ds(base, _EB)], rowbuf)
            descs = []
            for j in range(_NJ):
                descs.append(
                    pltpu.async_copy(
                        p_hbm.at[idxv.at[jj * _NJ + j]],
                        rowbuf.at[pl.ds(j * 128, 128)],
                        sem,
                        add=True,
                    )
                )
            for d in descs:
                d.wait()
            pltpu.sync_copy(rowbuf, z_hbm.at[pl.ds(base, _EB)])
        return carry

    lax.fori_loop(0, _IDXROWS_PER_W // 8, body, 0)


@functools.cache
def _make_sc_gather_add():
    return pl.kernel(
        _sc_gather_add_body,
        out_type=jax.ShapeDtypeStruct((NB_PAD, H), jnp.float32),
        mesh=_sc_mesh(),
        scratch_types=[
            pltpu.VMEM((_EB, H), jnp.float32),
            pltpu.VMEM((8, 128), jnp.int32),
            pltpu.SemaphoreType.DMA,
        ],
        interpret=False,
    )


_CH_SC = 5120
_CH = _CH_SC * NC
_NCHUNK = NB_PAD // _CH
_GARB = _CH_SC
_EROWS_PAD = 5120
_EROWS_PER_T = _EROWS_PAD // NS
_DRAIN = _CH_SC // NS


def _zero_zbuf(zbuf, nrows):
    def zrow(r, carry):
        for cc in range(8):
            zbuf[r, pl.ds(cc * 16, 16)] = jnp.zeros((16,), jnp.float32)
        return carry

    lax.fori_loop(0, nrows, zrow, 0)


def _sc_segsum_bb_body(
    h_hbm, bbs_hbm, bbd_hbm, m_hbm, rowbuf, srcv, lidx, zbuf, acc, sem, sem2
):
    k = lax.axis_index("c")
    s = lax.axis_index("s")
    erow0 = s * _EROWS_PER_T

    _zero_zbuf(zbuf, 32)
    for c in range(_NCHUNK):
        lo = c * _CH + k * _CH_SC
        for zz in range(_DRAIN // 32):
            pltpu.sync_copy(zbuf, acc.at[pl.ds(s * _DRAIN + zz * 32, 32)])
        plsc.subcore_barrier()

        def body(b, carry):
            er = erow0 + b * 8
            pltpu.sync_copy(bbd_hbm.at[pl.ds(er, 8)], lidx)
            pltpu.sync_copy(bbs_hbm.at[pl.ds(er, 8)], srcv)
            for j in range(8):
                for cc in range(8):
                    dd = lidx[j, pl.ds(cc * 16, 16)]
                    ss = srcv[j, pl.ds(cc * 16, 16)]
                    keep = (dd >= lo) & (dd < lo + _CH_SC)
                    neg = jnp.full((16,), -1, jnp.int32)
                    lidx[j, pl.ds(cc * 16, 16)] = jnp.where(keep, dd - lo, neg)
                    srcv[j, pl.ds(cc * 16, 16)] = jnp.where(keep, ss, neg)
            for jj in range(2):
                descs = []
                for j in range(_NJ):
                    descs.append(
                        pltpu.async_copy(
                            h_hbm.at[
                                plsc.Indices(
                                    srcv.at[jj * _NJ + j], ignored_value=-1
                                )
                            ],
                            rowbuf.at[pl.ds(j * 128, 128)],
                            sem,
                        )
                    )
                for d in descs:
                    d.wait()
                wdescs = []
                for j in range(_NJ):
                    wdescs.append(
                        pltpu.async_copy(
                            rowbuf.at[pl.ds(j * 128, 128)],
                            acc.at[
                                plsc.Indices(
                                    lidx.at[jj * _NJ + j], ignored_value=-1
                                )
                            ],
                            sem2,
                            add=True,
                        )
                    )
                for d in wdescs:
                    d.wait()
            return carry

        lax.fori_loop(0, _EROWS_PER_T // 8, body, 0)
        plsc.subcore_barrier()
        drain0 = c * _CH + k * _CH_SC + s * _DRAIN
        pltpu.sync_copy(
            acc.at[pl.ds(s * _DRAIN, _DRAIN)], m_hbm.at[pl.ds(drain0, _DRAIN)]
        )
        plsc.subcore_barrier()


@functools.cache
def _make_sc_segsum_bb():
    return pl.kernel(
        _sc_segsum_bb_body,
        out_type=jax.ShapeDtypeStruct((NB_PAD, H), jnp.float32),
        mesh=_sc_mesh(),
        scratch_types=[
            pltpu.VMEM((_EB, H), jnp.float32),
            pltpu.VMEM((8, 128), jnp.int32),
            pltpu.VMEM((8, 128), jnp.int32),
            pltpu.VMEM((32, H), jnp.float32),
            pltpu.VMEM_SHARED((_CH_SC + 8, H), jnp.float32),
            pltpu.SemaphoreType.DMA,
            pltpu.SemaphoreType.DMA,
        ],
        interpret=False,
    )


_AT_SC = 5120
_AT_PAD = _AT_SC * NC
_AT_DRAIN = _AT_SC // NS
_AT_ROWS_PER_T = NB_PAD // NS
_AT_IDXROWS_PER_T = _AT_ROWS_PER_T // 128


def _sc_segsum_atom_body(h_hbm, ba_hbm, mv_hbm, rowbuf, idxv, zbuf, acc, sem):
    k = lax.axis_index("c")
    s = lax.axis_index("s")
    row0 = s * _AT_ROWS_PER_T
    irow0 = s * _AT_IDXROWS_PER_T
    lo = k * _AT_SC

    _zero_zbuf(zbuf, 80)
    for zz in range(_AT_DRAIN // 80):
        pltpu.sync_copy(zbuf, acc.at[pl.ds(s * _AT_DRAIN + zz * 80, 80)])
    plsc.subcore_barrier()

    def body(b, carry):
        pltpu.sync_copy(ba_hbm.at[pl.ds(irow0 + b * 8, 8)], idxv)
        for j in range(8):
            for cc in range(8):
                a = idxv[j, pl.ds(cc * 16, 16)]
                keep = (a >= lo) & (a < lo + _AT_SC)
                idxv[j, pl.ds(cc * 16, 16)] = jnp.where(keep, a - lo, _AT_SC)
        for jj in range(2):
            base = row0 + (b * 2 + jj) * _EB
            pltpu.sync_copy(h_hbm.at[pl.ds(base, _EB)], rowbuf)
            for j in range(_NJ):
                pltpu.sync_copy(
                    rowbuf.at[pl.ds(j * 128, 128)],
                    acc.at[idxv.at[jj * _NJ + j]],
                    add=True,
                )
        return carry

    lax.fori_loop(0, _AT_IDXROWS_PER_T // 8, body, 0)
    plsc.subcore_barrier()
    pltpu.sync_copy(
        acc.at[pl.ds(s * _AT_DRAIN, _AT_DRAIN)],
        mv_hbm.at[pl.ds(lo + s * _AT_DRAIN, _AT_DRAIN)],
    )


@functools.cache
def _make_sc_segsum_atom():
    return pl.kernel(
        _sc_segsum_atom_body,
        out_type=jax.ShapeDtypeStruct((_AT_PAD, H), jnp.float32),
        mesh=_sc_mesh(),
        scratch_types=[
            pltpu.VMEM((_EB, H), jnp.float32),
            pltpu.VMEM((8, 128), jnp.int32),
            pltpu.VMEM((80, H), jnp.float32),
            pltpu.VMEM_SHARED((_AT_SC + 8, H), jnp.float32),
            pltpu.SemaphoreType.DMA,
        ],
        interpret=False,
    )


def kernel(atom_x, bond_e, W_i, W_h, W_o_w, W_o_b, src_atom, bb_src, bb_dst, bond_atom, mol_id):
    wa = W_i[:D_ATOM]
    wb = W_i[D_ATOM:]
    w_blk = jnp.zeros((128, 1024), jnp.float32)
    for j in range(8):
        w_blk = lax.dynamic_update_slice(w_blk, wb, (16 * j, 128 * j))

    src2d = jnp.pad(src_atom, (0, NB_PAD - N_BOND)).reshape(1280, 128)
    ba2d = jnp.pad(bond_atom, (0, NB_PAD - N_BOND), constant_values=N_ATOM).reshape(
        1280, 128
    )
    bbs2d = jnp.pad(bb_src, (0, _EROWS_PAD * 128 - E_BB)).reshape(_EROWS_PAD, 128)
    bbd2d = jnp.pad(
        bb_dst, (0, _EROWS_PAD * 128 - E_BB), constant_values=N_BOND
    ).reshape(_EROWS_PAD, 128)

    p = _tc_matmul(atom_x, wa, 2000)
    e2 = bond_e.reshape(20000, 128)
    bm = _tc_bond_proj(e2, w_blk).reshape(NB_PAD, H)

    z0 = _make_sc_gather_add()(p, bm, src2d)
    h0 = _tc_relu(z0)
    m = _make_sc_segsum_bb()(h0, bbs2d, bbd2d)
    h1 = _tc_h1(h0, m, W_h)
    mv = _make_sc_segsum_atom()(h1, ba2d)
    mol3d = mol_id.reshape(5, 1, 2000)
    out = _tc_final(
        atom_x, mv, W_o_w[:D_ATOM], W_o_w[D_ATOM:], W_o_b.reshape(1, H), mol3d
    )
    return out
